# baseline (speedup 1.0000x reference)
import jax
import jax.numpy as jnp
from jax.experimental import pallas as pl
from jax.experimental.pallas import tpu as pltpu

_B, _N = 8, 100000
_T = 100352
_INV_CNT = 1.0 / float(_B * _N)
_EPS = 1e-5
_D = 16
_C1, _C2, _C3 = 32, 32, 64


def _mlp_bn_max_kernel(pos_ref, feata_ref, featb_ref, w1_ref, w23_ref, gbe_ref,
                       out_ref, stat1, stat2, stat3, mx3, mn3,
                       w1ps, w1fs, c1s, w2s, c2s):
    p = pl.program_id(0)
    b = pl.program_id(1)
    seg_start = b == 0

    def dot(w, x):
        return jax.lax.dot_general(
            w, x, (((1,), (0,)), ((), ())),
            preferred_element_type=jnp.float32)

    def affine(stat_ref, g, be):
        m = stat_ref[:, 0:1] * _INV_CNT
        var = stat_ref[:, 1:2] * _INV_CNT - m * m
        a = g / jnp.sqrt(var + _EPS)
        return a, be - a * m

    @pl.when((p == 1) & seg_start)
    def _():
        a1, c1 = affine(stat1, gbe_ref[0:_C1], gbe_ref[_C1:2 * _C1])
        w1ps[...] = a1 * w1_ref[:, 0:3]
        w1fs[...] = a1 * w1_ref[:, 3:3 + _D]
        c1s[...] = c1

    @pl.when((p == 2) & seg_start)
    def _():
        a2, c2 = affine(stat2, gbe_ref[2 * _C1:2 * _C1 + _C2],
                        gbe_ref[2 * _C1 + _C2:2 * _C1 + 2 * _C2])
        w2s[...] = a2 * w23_ref[0:_C2, :]
        c2s[...] = c2

    def acc_stats(stat_ref, z):
        zv = z[:, :_N]
        s = jnp.sum(zv, axis=1, keepdims=True)
        q = jnp.sum(zv * zv, axis=1, keepdims=True)
        u = jnp.concatenate([s, q], axis=1)
        stat_ref[...] = jnp.where(seg_start, u, stat_ref[...] + u)

    _H = _D // 2

    def h1():
        z = (dot(w1ps[...], pos_ref[0])
             + dot(w1fs[:, :_H], feata_ref[0]) + dot(w1fs[:, _H:], featb_ref[0]))
        return jnp.maximum(z + c1s[...], 0.0)

    @pl.when(p == 0)
    def _():
        z1 = (dot(w1_ref[:, 0:3], pos_ref[0])
              + dot(w1_ref[:, 3:3 + _H], feata_ref[0])
              + dot(w1_ref[:, 3 + _H:3 + _D], featb_ref[0]))
        acc_stats(stat1, z1)

    @pl.when(p == 1)
    def _():
        acc_stats(stat2, dot(w23_ref[0:_C2, :], h1()))

    @pl.when(p == 2)
    def _():
        h2 = jnp.maximum(dot(w2s[...], h1()) + c2s[...], 0.0)
        z3 = dot(w23_ref[_C2:_C2 + _C3, :], h2)
        acc_stats(stat3, z3)

        z3v = z3[:, :_N]
        zmax = jnp.max(z3v, axis=1, keepdims=True)
        zmin = jnp.min(z3v, axis=1, keepdims=True)
        colm = jax.lax.broadcasted_iota(jnp.int32, (1, _B), 1) == b
        mxv = jnp.where(seg_start, -jnp.inf, mx3[...])
        mnv = jnp.where(seg_start, jnp.inf, mn3[...])
        mx3[...] = jnp.maximum(mxv, jnp.where(colm, zmax, -jnp.inf))
        mn3[...] = jnp.minimum(mnv, jnp.where(colm, zmin, jnp.inf))

        @pl.when(b == _B - 1)
        def _():
            base = 2 * _C1 + 2 * _C2
            a3, c3 = affine(stat3, gbe_ref[base:base + _C3],
                            gbe_ref[base + _C3:base + 2 * _C3])
            pick = jnp.where(a3 >= 0.0, mx3[...], mn3[...])
            out_ref[...] = jnp.maximum(a3 * pick + c3, 0.0)


def kernel(points_position, points_feature, W1, b1, g1, be1,
           W2, b2, g2, be2, W3, b3, g3, be3):
    B, _, N = points_position.shape
    D = points_feature.shape[1]
    del b1, b2, b3
    w23 = jnp.concatenate([W2, W3], axis=0)
    gbe = jnp.concatenate([g1, be1, g2, be2, g3, be3])[:, None]
    c3 = W3.shape[0]

    const = lambda p_, b_: (0, 0)
    out = pl.pallas_call(
        _mlp_bn_max_kernel,
        grid=(3, _B),
        in_specs=[
            pl.BlockSpec((1, 3, _T), lambda p_, b_: (b_, 0, 0)),
            pl.BlockSpec((1, D // 2, _T), lambda p_, b_: (b_, 0, 0)),
            pl.BlockSpec((1, D // 2, _T), lambda p_, b_: (b_, 1, 0)),
            pl.BlockSpec(W1.shape, const),
            pl.BlockSpec(w23.shape, const),
            pl.BlockSpec(gbe.shape, const),
        ],
        out_specs=pl.BlockSpec((c3, _B), const),
        out_shape=jax.ShapeDtypeStruct((c3, _B), jnp.float32),
        scratch_shapes=[
            pltpu.VMEM((_C1, 2), jnp.float32),
            pltpu.VMEM((_C2, 2), jnp.float32),
            pltpu.VMEM((_C3, 2), jnp.float32),
            pltpu.VMEM((_C3, _B), jnp.float32),
            pltpu.VMEM((_C3, _B), jnp.float32),
            pltpu.VMEM((_C1, 3), jnp.float32),
            pltpu.VMEM((_C1, _D), jnp.float32),
            pltpu.VMEM((_C1, 1), jnp.float32),
            pltpu.VMEM((_C2, _C1), jnp.float32),
            pltpu.VMEM((_C2, 1), jnp.float32),
        ],
        compiler_params=pltpu.CompilerParams(
            dimension_semantics=("arbitrary", "arbitrary")),
    )(points_position, points_feature, points_feature, W1, w23, gbe)

    feat_out = out.T[:, :, None]
    pos_out = jnp.zeros((B, 3, 1), dtype=points_position.dtype)
    return (pos_out, feat_out)


# speedup vs baseline: 1.2237x; 1.2237x over previous
import jax
import jax.numpy as jnp
from jax.experimental import pallas as pl
from jax.experimental.pallas import tpu as pltpu

_B, _N = 8, 100000
_T = 100352
_INV_CNT = 1.0 / float(_B * _N)
_EPS = 1e-5
_D = 16
_C1, _C2, _C3 = 32, 32, 64


def _mlp_bn_max_kernel(pos_ref, feat_ref, w1_ref, w23_ref, gbe_ref,
                       out_ref, stat1, stat2, stat3, mx3, mn3,
                       w1ps, w1fs, c1s, w2s, c2s):
    p = pl.program_id(0)
    b = pl.program_id(1)
    seg_start = b == 0

    def dot(w, x):
        return jax.lax.dot_general(
            w, x, (((1,), (0,)), ((), ())),
            preferred_element_type=jnp.float32)

    def affine(stat_ref, g, be):
        m = stat_ref[:, 0:1] * _INV_CNT
        var = stat_ref[:, 1:2] * _INV_CNT - m * m
        a = g / jnp.sqrt(var + _EPS)
        return a, be - a * m

    @pl.when((p == 1) & seg_start)
    def _():
        a1, c1 = affine(stat1, gbe_ref[0:_C1], gbe_ref[_C1:2 * _C1])
        w1ps[...] = a1 * w1_ref[:, 0:3]
        w1fs[...] = a1 * w1_ref[:, 3:3 + _D]
        c1s[...] = c1

    @pl.when((p == 2) & seg_start)
    def _():
        a2, c2 = affine(stat2, gbe_ref[2 * _C1:2 * _C1 + _C2],
                        gbe_ref[2 * _C1 + _C2:2 * _C1 + 2 * _C2])
        w2s[...] = a2 * w23_ref[0:_C2, :]
        c2s[...] = c2

    _S = 4
    _TS = _T // _S

    def sub(ref, i):
        return ref[0][:, i * _TS:(i + 1) * _TS]

    def valid(z, i):
        lo, hi = i * _TS, min((i + 1) * _TS, _N)
        return z[:, :hi - lo]

    def commit_stats(stat_ref, parts):
        s = sum(jnp.sum(zv, axis=1, keepdims=True) for zv in parts)
        q = sum(jnp.sum(zv * zv, axis=1, keepdims=True) for zv in parts)
        u = jnp.concatenate([s, q], axis=1)
        stat_ref[...] = jnp.where(seg_start, u, stat_ref[...] + u)

    def h1(i):
        z = dot(w1ps[...], sub(pos_ref, i)) + dot(w1fs[...], sub(feat_ref, i))
        return jnp.maximum(z + c1s[...], 0.0)

    @pl.when(p == 0)
    def _():
        parts = []
        for i in range(_S):
            z1 = (dot(w1_ref[:, 0:3], sub(pos_ref, i))
                  + dot(w1_ref[:, 3:3 + _D], sub(feat_ref, i)))
            parts.append(valid(z1, i))
        commit_stats(stat1, parts)

    @pl.when(p == 1)
    def _():
        parts = []
        for i in range(_S):
            z2 = dot(w23_ref[0:_C2, :], h1(i))
            parts.append(valid(z2, i))
        commit_stats(stat2, parts)

    @pl.when(p == 2)
    def _():
        parts, mxp, mnp = [], [], []
        for i in range(_S):
            h2 = jnp.maximum(dot(w2s[...], h1(i)) + c2s[...], 0.0)
            z3 = dot(w23_ref[_C2:_C2 + _C3, :], h2)
            z3v = valid(z3, i)
            parts.append(z3v)
            mxp.append(jnp.max(z3v, axis=1, keepdims=True))
            mnp.append(jnp.min(z3v, axis=1, keepdims=True))
        commit_stats(stat3, parts)

        zmax = jnp.maximum(jnp.maximum(mxp[0], mxp[1]), jnp.maximum(mxp[2], mxp[3]))
        zmin = jnp.minimum(jnp.minimum(mnp[0], mnp[1]), jnp.minimum(mnp[2], mnp[3]))
        colm = jax.lax.broadcasted_iota(jnp.int32, (1, _B), 1) == b
        mxv = jnp.where(seg_start, -jnp.inf, mx3[...])
        mnv = jnp.where(seg_start, jnp.inf, mn3[...])
        mx3[...] = jnp.maximum(mxv, jnp.where(colm, zmax, -jnp.inf))
        mn3[...] = jnp.minimum(mnv, jnp.where(colm, zmin, jnp.inf))

        @pl.when(b == _B - 1)
        def _():
            base = 2 * _C1 + 2 * _C2
            a3, c3 = affine(stat3, gbe_ref[base:base + _C3],
                            gbe_ref[base + _C3:base + 2 * _C3])
            pick = jnp.where(a3 >= 0.0, mx3[...], mn3[...])
            out_ref[...] = jnp.maximum(a3 * pick + c3, 0.0)


def kernel(points_position, points_feature, W1, b1, g1, be1,
           W2, b2, g2, be2, W3, b3, g3, be3):
    B, _, N = points_position.shape
    D = points_feature.shape[1]
    del b1, b2, b3
    w23 = jnp.concatenate([W2, W3], axis=0)
    gbe = jnp.concatenate([g1, be1, g2, be2, g3, be3])[:, None]
    c3 = W3.shape[0]

    const = lambda p_, b_: (0, 0)
    out = pl.pallas_call(
        _mlp_bn_max_kernel,
        grid=(3, _B),
        in_specs=[
            pl.BlockSpec((1, 3, _T), lambda p_, b_: (b_, 0, 0)),
            pl.BlockSpec((1, D, _T), lambda p_, b_: (b_, 0, 0)),
            pl.BlockSpec(W1.shape, const),
            pl.BlockSpec(w23.shape, const),
            pl.BlockSpec(gbe.shape, const),
        ],
        out_specs=pl.BlockSpec((c3, _B), const),
        out_shape=jax.ShapeDtypeStruct((c3, _B), jnp.float32),
        scratch_shapes=[
            pltpu.VMEM((_C1, 2), jnp.float32),
            pltpu.VMEM((_C2, 2), jnp.float32),
            pltpu.VMEM((_C3, 2), jnp.float32),
            pltpu.VMEM((_C3, _B), jnp.float32),
            pltpu.VMEM((_C3, _B), jnp.float32),
            pltpu.VMEM((_C1, 3), jnp.float32),
            pltpu.VMEM((_C1, _D), jnp.float32),
            pltpu.VMEM((_C1, 1), jnp.float32),
            pltpu.VMEM((_C2, _C1), jnp.float32),
            pltpu.VMEM((_C2, 1), jnp.float32),
        ],
        compiler_params=pltpu.CompilerParams(
            dimension_semantics=("arbitrary", "arbitrary")),
    )(points_position, points_feature, W1, w23, gbe)

    feat_out = out.T[:, :, None]
    pos_out = jnp.zeros((B, 3, 1), dtype=points_position.dtype)
    return (pos_out, feat_out)
